# baseline (speedup 1.0000x reference)
import functools

import jax
import jax.numpy as jnp
from jax import lax
from jax.experimental import pallas as pl
from jax.experimental.pallas import tpu as pltpu
from jax.experimental.pallas import tpu_sc as plsc

_VT = 8192


def _sc_gather_all(emb_in2, emb_tg2, w_lin2, b_lin2,
                   idx_enc, idx_dec, idx_tgt, idx_bhi):
    info = plsc.get_sparse_core_info()
    nw = info.num_cores * info.num_subcores
    nc = info.num_cores
    n_enc = idx_enc.shape[0]
    n_dec = idx_dec.shape[0]
    d = emb_in2.shape[1]
    ce = n_enc // nw
    cd = n_dec // nw

    mesh = plsc.VectorSubcoreMesh(core_axis_name="c", subcore_axis_name="s")

    @functools.partial(
        pl.kernel,
        mesh=mesh,
        out_type=[
            jax.ShapeDtypeStruct((n_enc, d), jnp.float32),
            jax.ShapeDtypeStruct((n_dec, d), jnp.float32),
            jax.ShapeDtypeStruct((n_dec, d), jnp.float32),
            jax.ShapeDtypeStruct((n_dec, d), jnp.float32),
        ],
        scratch_types=[
            pltpu.VMEM((ce,), jnp.int32),
            pltpu.VMEM((ce, d), jnp.float32),
            pltpu.SemaphoreType.DMA,
        ],
    )
    def gather_kernel(emb_in_h, emb_tg_h, w_lin_h, b_lin_h,
                      idx_enc_h, idx_dec_h, idx_tgt_h, idx_bhi_h,
                      enc_o, dec_o, wr_o, br_o,
                      idx_v, row_v, sem):
        wid = lax.axis_index("s") * nc + lax.axis_index("c")
        be = wid * ce
        bd = wid * cd

        def gather_one(idx_h, tab_h, out_h, base, n):
            pltpu.sync_copy(idx_h.at[pl.ds(base, n)], idx_v.at[pl.ds(0, n)])
            pltpu.async_copy(tab_h.at[idx_v.at[pl.ds(0, n)]],
                             row_v.at[pl.ds(0, n)], sem).wait()
            pltpu.sync_copy(row_v.at[pl.ds(0, n)], out_h.at[pl.ds(base, n)])

        gather_one(idx_enc_h, emb_in_h, enc_o, be, ce)
        gather_one(idx_dec_h, emb_tg_h, dec_o, bd, cd)
        gather_one(idx_tgt_h, w_lin_h, wr_o, bd, cd)
        gather_one(idx_bhi_h, b_lin_h, br_o, bd, cd)

    return gather_kernel(emb_in2, emb_tg2, w_lin2, b_lin2,
                         idx_enc, idx_dec, idx_tgt, idx_bhi)


def _lstm_body(enc_ref, dec_ref, wr_ref, br_ref,
               pe_ref, pd_ref, pt_ref, pb_ref,
               wih_i_ref, whh_i_ref, bi_ref, wih_t_ref, whh_t_ref, bt_ref,
               h_out_ref, tsum_ref, *, n_enc, n_dec, batch, hd):

    def half(row2, par):
        sel = (par[:, None] == 1).astype(jnp.float32)
        return row2[:, hd:] * sel + row2[:, :hd] * (1.0 - sel)

    def cell(x, h, c, wih, whh, b):
        gates = (jnp.dot(x, wih, preferred_element_type=jnp.float32)
                 + jnp.dot(h, whh, preferred_element_type=jnp.float32) + b)
        i = jax.nn.sigmoid(gates[:, 0 * hd:1 * hd])
        f = jax.nn.sigmoid(gates[:, 1 * hd:2 * hd])
        g = jnp.tanh(gates[:, 2 * hd:3 * hd])
        o = jax.nn.sigmoid(gates[:, 3 * hd:4 * hd])
        c = f * c + i * g
        h = o * jnp.tanh(c)
        return h, c

    wih_i = wih_i_ref[:]
    whh_i = whh_i_ref[:]
    bi = bi_ref[:]
    wih_t = wih_t_ref[:]
    whh_t = whh_t_ref[:]
    bt = bt_ref[:]

    def enc_step(t, carry):
        h, c = carry
        x = half(enc_ref[pl.ds(t * batch, batch), :],
                 pe_ref[pl.ds(t * batch, batch)])
        return cell(x, h, c, wih_i, whh_i, bi)

    z = jnp.zeros((batch, hd), dtype=jnp.float32)
    h, c = lax.fori_loop(0, n_enc, enc_step, (z, z))

    def dec_step(t, carry):
        h, c, acc = carry
        x = half(dec_ref[pl.ds(t * batch, batch), :],
                 pd_ref[pl.ds(t * batch, batch)])
        h, c = cell(x, h, c, wih_t, whh_t, bt)
        h_out_ref[pl.ds(t * batch, batch), :] = h.astype(jnp.bfloat16)
        w_row = half(wr_ref[pl.ds(t * batch, batch), :],
                     pt_ref[pl.ds(t * batch, batch)])
        onehot = (lax.broadcasted_iota(jnp.int32, (batch, 128), 1)
                  == pb_ref[pl.ds(t * batch, batch)][:, None]
                  ).astype(jnp.float32)
        tl = (jnp.sum(h * w_row)
              + jnp.sum(br_ref[pl.ds(t * batch, batch), :] * onehot))
        return h, c, acc + tl

    h, c, acc = lax.fori_loop(0, n_dec, dec_step,
                              (h, c, jnp.float32(0.0)))
    tsum_ref[:] = jnp.reshape(acc, (1, 1))


def _run_lstm(enc_g, dec_g, wr_g, br_g, pe, pd, pt, pb,
              wih_i, whh_i, bi, wih_t, whh_t, bt, n_enc, n_dec, batch, hd):
    body = functools.partial(_lstm_body, n_enc=n_enc, n_dec=n_dec,
                             batch=batch, hd=hd)
    return pl.pallas_call(
        body,
        out_shape=[
            jax.ShapeDtypeStruct((n_dec * batch, hd), jnp.bfloat16),
            jax.ShapeDtypeStruct((1, 1), jnp.float32),
        ],
    )(enc_g, dec_g, wr_g, br_g, pe, pd, pt, pb,
      wih_i, whh_i, bi, wih_t, whh_t, bt)


def _lse_body(h_ref, w_ref, b_ref, tsum_ref, out_ref, s_ref, *, num_v, rt,
              inv_batch):
    v = pl.program_id(0)
    r = pl.program_id(1)
    logits = lax.dot_general(
        w_ref[:], h_ref[:], (((1,), (1,)), ((), ())),
        preferred_element_type=jnp.float32)
    part = jnp.sum(jnp.exp2(logits + b_ref[:]), axis=0).reshape(rt // 128, 128)

    @pl.when(jnp.logical_and(v == 0, r == 0))
    def _():
        out_ref[:] = -tsum_ref[:] * inv_batch

    @pl.when(v == 0)
    def _():
        s_ref[r] = part

    @pl.when(v > 0)
    def _():
        s_ref[r] = s_ref[r] + part

    @pl.when(v == num_v - 1)
    def _():
        lse = jnp.log(s_ref[r])
        out_ref[:] = out_ref[:] + jnp.sum(lse, keepdims=True).reshape(1, 1) * inv_batch


def _run_lse(h_bf, w_bf, b_pad, tsum, batch):
    n_rows = h_bf.shape[0]
    vp = w_bf.shape[0]
    vt = _VT
    rt = 1024
    num_v = vp // vt
    num_r = n_rows // rt
    body = functools.partial(_lse_body, num_v=num_v, rt=rt,
                             inv_batch=1.0 / batch)
    return pl.pallas_call(
        body,
        grid=(num_v, num_r),
        in_specs=[
            pl.BlockSpec((rt, h_bf.shape[1]), lambda v, r: (r, 0)),
            pl.BlockSpec((vt, w_bf.shape[1]), lambda v, r: (v, 0)),
            pl.BlockSpec((vt, 1), lambda v, r: (v, 0)),
            pl.BlockSpec((1, 1), lambda v, r: (0, 0)),
        ],
        out_specs=pl.BlockSpec((1, 1), lambda v, r: (0, 0)),
        out_shape=jax.ShapeDtypeStruct((1, 1), jnp.float32),
        scratch_shapes=[pltpu.VMEM((num_r, rt // 128, 128), jnp.float32)],
    )(h_bf, w_bf, b_pad, tsum)


def kernel(input_lines, target_lines, embed_input, embed_target,
           W_ih_in, W_hh_in, b_ih_in, b_hh_in,
           W_ih_tg, W_hh_tg, b_ih_tg, b_hh_tg,
           W_lin, b_lin):
    s_in, batch = input_lines.shape
    s_out = target_lines.shape[0]
    t_dec = s_out - 1
    hd = embed_input.shape[1]
    v = W_lin.shape[0]

    idx_enc = input_lines.reshape(-1)
    idx_dec = target_lines[:t_dec].reshape(-1)
    idx_tgt = target_lines[1:].reshape(-1)

    emb_in2 = embed_input.reshape(embed_input.shape[0] // 2, 2 * hd)
    emb_tg2 = embed_target.reshape(embed_target.shape[0] // 2, 2 * hd)
    w_lin2 = W_lin.reshape(v // 2, 2 * hd)
    b_pad128 = jnp.pad(b_lin, (0, (-v) % 128))
    b_lin2 = b_pad128.reshape(b_pad128.shape[0] // 128, 128)

    pe = jnp.bitwise_and(idx_enc, 1)
    pd = jnp.bitwise_and(idx_dec, 1)
    pt = jnp.bitwise_and(idx_tgt, 1)
    pb = jnp.bitwise_and(idx_tgt, 127)

    enc_g, dec_g, wr_g, br_g = _sc_gather_all(
        emb_in2, emb_tg2, w_lin2, b_lin2,
        lax.shift_right_logical(idx_enc, 1),
        lax.shift_right_logical(idx_dec, 1),
        lax.shift_right_logical(idx_tgt, 1),
        lax.shift_right_logical(idx_tgt, 7))

    h_bf, tsum = _run_lstm(
        enc_g, dec_g, wr_g, br_g, pe, pd, pt, pb,
        W_ih_in.T, W_hh_in.T, (b_ih_in + b_hh_in)[None, :],
        W_ih_tg.T, W_hh_tg.T, (b_ih_tg + b_hh_tg)[None, :],
        s_in, t_dec, batch, hd)

    vt = _VT
    vp = ((v + vt - 1) // vt) * vt
    log2e = 1.4426950408889634
    w_bf = jnp.pad((W_lin * log2e).astype(jnp.bfloat16), ((0, vp - v), (0, 0)))
    b_pad = jnp.pad(b_lin * log2e, (0, vp - v),
                    constant_values=-1e30).reshape(vp, 1)

    loss = _run_lse(h_bf, w_bf, b_pad, tsum, batch)
    return loss.reshape(())


# speedup vs baseline: 1.0189x; 1.0189x over previous
import functools

import jax
import jax.numpy as jnp
from jax import lax
from jax.experimental import pallas as pl
from jax.experimental.pallas import tpu as pltpu
from jax.experimental.pallas import tpu_sc as plsc

_VT = 6400


def _sc_gather_all(emb_in2, emb_tg2, w_lin2, b_lin2,
                   idx_enc, idx_dec, idx_tgt, idx_bhi):
    info = plsc.get_sparse_core_info()
    nw = info.num_cores * info.num_subcores
    nc = info.num_cores
    n_enc = idx_enc.shape[0]
    n_dec = idx_dec.shape[0]
    d = emb_in2.shape[1]
    ce = n_enc // nw
    cd = n_dec // nw

    mesh = plsc.VectorSubcoreMesh(core_axis_name="c", subcore_axis_name="s")

    @functools.partial(
        pl.kernel,
        mesh=mesh,
        out_type=[
            jax.ShapeDtypeStruct((n_enc, d), jnp.float32),
            jax.ShapeDtypeStruct((n_dec, d), jnp.float32),
            jax.ShapeDtypeStruct((n_dec, d), jnp.float32),
            jax.ShapeDtypeStruct((n_dec, d), jnp.float32),
        ],
        scratch_types=[
            pltpu.VMEM((ce,), jnp.int32),
            pltpu.VMEM((ce, d), jnp.float32),
            pltpu.SemaphoreType.DMA,
        ],
    )
    def gather_kernel(emb_in_h, emb_tg_h, w_lin_h, b_lin_h,
                      idx_enc_h, idx_dec_h, idx_tgt_h, idx_bhi_h,
                      enc_o, dec_o, wr_o, br_o,
                      idx_v, row_v, sem):
        wid = lax.axis_index("s") * nc + lax.axis_index("c")
        be = wid * ce
        bd = wid * cd

        def gather_one(idx_h, tab_h, out_h, base, n):
            pltpu.sync_copy(idx_h.at[pl.ds(base, n)], idx_v.at[pl.ds(0, n)])
            pltpu.async_copy(tab_h.at[idx_v.at[pl.ds(0, n)]],
                             row_v.at[pl.ds(0, n)], sem).wait()
            pltpu.sync_copy(row_v.at[pl.ds(0, n)], out_h.at[pl.ds(base, n)])

        gather_one(idx_enc_h, emb_in_h, enc_o, be, ce)
        gather_one(idx_dec_h, emb_tg_h, dec_o, bd, cd)
        gather_one(idx_tgt_h, w_lin_h, wr_o, bd, cd)
        gather_one(idx_bhi_h, b_lin_h, br_o, bd, cd)

    return gather_kernel(emb_in2, emb_tg2, w_lin2, b_lin2,
                         idx_enc, idx_dec, idx_tgt, idx_bhi)


def _lstm_body(enc_ref, dec_ref, wr_ref, br_ref,
               pe_ref, pd_ref, pt_ref, pb_ref,
               wih_i_ref, whh_i_ref, bi_ref, wih_t_ref, whh_t_ref, bt_ref,
               h_out_ref, tsum_ref, *, n_enc, n_dec, batch, hd):

    def half(row2, par):
        sel = (par[:, None] == 1).astype(jnp.float32)
        return row2[:, hd:] * sel + row2[:, :hd] * (1.0 - sel)

    def cell(x, h, c, wih, whh, b):
        gates = (jnp.dot(x, wih, preferred_element_type=jnp.float32)
                 + jnp.dot(h, whh, preferred_element_type=jnp.float32) + b)
        i = jax.nn.sigmoid(gates[:, 0 * hd:1 * hd])
        f = jax.nn.sigmoid(gates[:, 1 * hd:2 * hd])
        g = jnp.tanh(gates[:, 2 * hd:3 * hd])
        o = jax.nn.sigmoid(gates[:, 3 * hd:4 * hd])
        c = f * c + i * g
        h = o * jnp.tanh(c)
        return h, c

    wih_i = wih_i_ref[:]
    whh_i = whh_i_ref[:]
    bi = bi_ref[:]
    wih_t = wih_t_ref[:]
    whh_t = whh_t_ref[:]
    bt = bt_ref[:]

    def enc_step(t, carry):
        h, c = carry
        x = half(enc_ref[pl.ds(t * batch, batch), :],
                 pe_ref[pl.ds(t * batch, batch)])
        return cell(x, h, c, wih_i, whh_i, bi)

    z = jnp.zeros((batch, hd), dtype=jnp.float32)
    h, c = lax.fori_loop(0, n_enc, enc_step, (z, z))

    def dec_step(t, carry):
        h, c, acc = carry
        x = half(dec_ref[pl.ds(t * batch, batch), :],
                 pd_ref[pl.ds(t * batch, batch)])
        h, c = cell(x, h, c, wih_t, whh_t, bt)
        h_out_ref[pl.ds(t * batch, batch), :] = h.astype(jnp.bfloat16)
        w_row = half(wr_ref[pl.ds(t * batch, batch), :],
                     pt_ref[pl.ds(t * batch, batch)])
        onehot = (lax.broadcasted_iota(jnp.int32, (batch, 128), 1)
                  == pb_ref[pl.ds(t * batch, batch)][:, None]
                  ).astype(jnp.float32)
        tl = (jnp.sum(h * w_row)
              + jnp.sum(br_ref[pl.ds(t * batch, batch), :] * onehot))
        return h, c, acc + tl

    h, c, acc = lax.fori_loop(0, n_dec, dec_step,
                              (h, c, jnp.float32(0.0)))
    tsum_ref[:] = jnp.reshape(acc, (1, 1))


def _run_lstm(enc_g, dec_g, wr_g, br_g, pe, pd, pt, pb,
              wih_i, whh_i, bi, wih_t, whh_t, bt, n_enc, n_dec, batch, hd):
    body = functools.partial(_lstm_body, n_enc=n_enc, n_dec=n_dec,
                             batch=batch, hd=hd)
    return pl.pallas_call(
        body,
        out_shape=[
            jax.ShapeDtypeStruct((n_dec * batch, hd), jnp.bfloat16),
            jax.ShapeDtypeStruct((1, 1), jnp.float32),
        ],
    )(enc_g, dec_g, wr_g, br_g, pe, pd, pt, pb,
      wih_i, whh_i, bi, wih_t, whh_t, bt)


def _lse_body(h_ref, w_ref, b_ref, tsum_ref, out_ref, s_ref, *, num_v, rt,
              inv_batch):
    v = pl.program_id(0)
    r = pl.program_id(1)
    logits = lax.dot_general(
        w_ref[:], h_ref[:], (((1,), (1,)), ((), ())),
        preferred_element_type=jnp.float32)
    part = jnp.sum(jnp.exp2(logits + b_ref[:]), axis=0).reshape(rt // 128, 128)

    @pl.when(jnp.logical_and(v == 0, r == 0))
    def _():
        out_ref[:] = -tsum_ref[:] * inv_batch

    @pl.when(v == 0)
    def _():
        s_ref[r] = part

    @pl.when(v > 0)
    def _():
        s_ref[r] = s_ref[r] + part

    @pl.when(v == num_v - 1)
    def _():
        lse = jnp.log(s_ref[r])
        out_ref[:] = out_ref[:] + jnp.sum(lse, keepdims=True).reshape(1, 1) * inv_batch


def _run_lse(h_bf, w_bf, b_pad, tsum, batch):
    n_rows = h_bf.shape[0]
    vp = w_bf.shape[0]
    vt = _VT
    rt = 1024
    num_v = vp // vt
    num_r = n_rows // rt
    body = functools.partial(_lse_body, num_v=num_v, rt=rt,
                             inv_batch=1.0 / batch)
    return pl.pallas_call(
        body,
        grid=(num_v, num_r),
        in_specs=[
            pl.BlockSpec((rt, h_bf.shape[1]), lambda v, r: (r, 0)),
            pl.BlockSpec((vt, w_bf.shape[1]), lambda v, r: (v, 0)),
            pl.BlockSpec((vt, 1), lambda v, r: (v, 0)),
            pl.BlockSpec((1, 1), lambda v, r: (0, 0)),
        ],
        out_specs=pl.BlockSpec((1, 1), lambda v, r: (0, 0)),
        out_shape=jax.ShapeDtypeStruct((1, 1), jnp.float32),
        scratch_shapes=[pltpu.VMEM((num_r, rt // 128, 128), jnp.float32)],
    )(h_bf, w_bf, b_pad, tsum)


def kernel(input_lines, target_lines, embed_input, embed_target,
           W_ih_in, W_hh_in, b_ih_in, b_hh_in,
           W_ih_tg, W_hh_tg, b_ih_tg, b_hh_tg,
           W_lin, b_lin):
    s_in, batch = input_lines.shape
    s_out = target_lines.shape[0]
    t_dec = s_out - 1
    hd = embed_input.shape[1]
    v = W_lin.shape[0]

    idx_enc = input_lines.reshape(-1)
    idx_dec = target_lines[:t_dec].reshape(-1)
    idx_tgt = target_lines[1:].reshape(-1)

    emb_in2 = embed_input.reshape(embed_input.shape[0] // 2, 2 * hd)
    emb_tg2 = embed_target.reshape(embed_target.shape[0] // 2, 2 * hd)
    w_lin2 = W_lin.reshape(v // 2, 2 * hd)
    b_pad128 = jnp.pad(b_lin, (0, (-v) % 128))
    b_lin2 = b_pad128.reshape(b_pad128.shape[0] // 128, 128)

    pe = jnp.bitwise_and(idx_enc, 1)
    pd = jnp.bitwise_and(idx_dec, 1)
    pt = jnp.bitwise_and(idx_tgt, 1)
    pb = jnp.bitwise_and(idx_tgt, 127)

    enc_g, dec_g, wr_g, br_g = _sc_gather_all(
        emb_in2, emb_tg2, w_lin2, b_lin2,
        lax.shift_right_logical(idx_enc, 1),
        lax.shift_right_logical(idx_dec, 1),
        lax.shift_right_logical(idx_tgt, 1),
        lax.shift_right_logical(idx_tgt, 7))

    h_bf, tsum = _run_lstm(
        enc_g, dec_g, wr_g, br_g, pe, pd, pt, pb,
        W_ih_in.T, W_hh_in.T, (b_ih_in + b_hh_in)[None, :],
        W_ih_tg.T, W_hh_tg.T, (b_ih_tg + b_hh_tg)[None, :],
        s_in, t_dec, batch, hd)

    vt = _VT
    vp = ((v + vt - 1) // vt) * vt
    log2e = 1.4426950408889634
    w_bf = jnp.pad((W_lin * log2e).astype(jnp.bfloat16), ((0, vp - v), (0, 0)))
    b_pad = jnp.pad(b_lin * log2e, (0, vp - v),
                    constant_values=-1e30).reshape(vp, 1)

    loss = _run_lse(h_bf, w_bf, b_pad, tsum, batch)
    return loss.reshape(())


# speedup vs baseline: 1.0394x; 1.0201x over previous
import functools

import jax
import jax.numpy as jnp
from jax import lax
from jax.experimental import pallas as pl
from jax.experimental.pallas import tpu as pltpu
from jax.experimental.pallas import tpu_sc as plsc

_VT = 10240


def _sc_gather_all(emb_in2, emb_tg2, w_lin2, b_lin2,
                   idx_enc, idx_dec, idx_tgt, idx_bhi):
    info = plsc.get_sparse_core_info()
    nw = info.num_cores * info.num_subcores
    nc = info.num_cores
    n_enc = idx_enc.shape[0]
    n_dec = idx_dec.shape[0]
    d = emb_in2.shape[1]
    ce = n_enc // nw
    cd = n_dec // nw

    mesh = plsc.VectorSubcoreMesh(core_axis_name="c", subcore_axis_name="s")

    @functools.partial(
        pl.kernel,
        mesh=mesh,
        out_type=[
            jax.ShapeDtypeStruct((n_enc, d), jnp.float32),
            jax.ShapeDtypeStruct((n_dec, d), jnp.float32),
            jax.ShapeDtypeStruct((n_dec, d), jnp.float32),
            jax.ShapeDtypeStruct((n_dec, d), jnp.float32),
        ],
        scratch_types=[
            pltpu.VMEM((ce,), jnp.int32),
            pltpu.VMEM((ce, d), jnp.float32),
            pltpu.SemaphoreType.DMA,
        ],
    )
    def gather_kernel(emb_in_h, emb_tg_h, w_lin_h, b_lin_h,
                      idx_enc_h, idx_dec_h, idx_tgt_h, idx_bhi_h,
                      enc_o, dec_o, wr_o, br_o,
                      idx_v, row_v, sem):
        wid = lax.axis_index("s") * nc + lax.axis_index("c")
        be = wid * ce
        bd = wid * cd

        def gather_one(idx_h, tab_h, out_h, base, n):
            pltpu.sync_copy(idx_h.at[pl.ds(base, n)], idx_v.at[pl.ds(0, n)])
            pltpu.async_copy(tab_h.at[idx_v.at[pl.ds(0, n)]],
                             row_v.at[pl.ds(0, n)], sem).wait()
            pltpu.sync_copy(row_v.at[pl.ds(0, n)], out_h.at[pl.ds(base, n)])

        gather_one(idx_enc_h, emb_in_h, enc_o, be, ce)
        gather_one(idx_dec_h, emb_tg_h, dec_o, bd, cd)
        gather_one(idx_tgt_h, w_lin_h, wr_o, bd, cd)
        gather_one(idx_bhi_h, b_lin_h, br_o, bd, cd)

    return gather_kernel(emb_in2, emb_tg2, w_lin2, b_lin2,
                         idx_enc, idx_dec, idx_tgt, idx_bhi)


def _lstm_body(enc_ref, dec_ref, wr_ref, br_ref,
               pe_ref, pd_ref, pt_ref, pb_ref,
               wih_i_ref, whh_i_ref, bi_ref, wih_t_ref, whh_t_ref, bt_ref,
               h_out_ref, tsum_ref, *, n_enc, n_dec, batch, hd):

    def half(row2, par):
        sel = (par[:, None] == 1).astype(jnp.float32)
        return row2[:, hd:] * sel + row2[:, :hd] * (1.0 - sel)

    def cell(x, h, c, wih, whh, b):
        gates = (jnp.dot(x, wih, preferred_element_type=jnp.float32)
                 + jnp.dot(h, whh, preferred_element_type=jnp.float32) + b)
        i = jax.nn.sigmoid(gates[:, 0 * hd:1 * hd])
        f = jax.nn.sigmoid(gates[:, 1 * hd:2 * hd])
        g = jnp.tanh(gates[:, 2 * hd:3 * hd])
        o = jax.nn.sigmoid(gates[:, 3 * hd:4 * hd])
        c = f * c + i * g
        h = o * jnp.tanh(c)
        return h, c

    wih_i = wih_i_ref[:]
    whh_i = whh_i_ref[:]
    bi = bi_ref[:]
    wih_t = wih_t_ref[:]
    whh_t = whh_t_ref[:]
    bt = bt_ref[:]

    def enc_step(t, carry):
        h, c = carry
        x = half(enc_ref[pl.ds(t * batch, batch), :],
                 pe_ref[pl.ds(t * batch, batch)])
        return cell(x, h, c, wih_i, whh_i, bi)

    z = jnp.zeros((batch, hd), dtype=jnp.float32)
    h, c = lax.fori_loop(0, n_enc, enc_step, (z, z))

    def dec_step(t, carry):
        h, c, acc = carry
        x = half(dec_ref[pl.ds(t * batch, batch), :],
                 pd_ref[pl.ds(t * batch, batch)])
        h, c = cell(x, h, c, wih_t, whh_t, bt)
        h_out_ref[pl.ds(t * batch, batch), :] = h.astype(jnp.bfloat16)
        w_row = half(wr_ref[pl.ds(t * batch, batch), :],
                     pt_ref[pl.ds(t * batch, batch)])
        onehot = (lax.broadcasted_iota(jnp.int32, (batch, 128), 1)
                  == pb_ref[pl.ds(t * batch, batch)][:, None]
                  ).astype(jnp.float32)
        tl = (jnp.sum(h * w_row)
              + jnp.sum(br_ref[pl.ds(t * batch, batch), :] * onehot))
        return h, c, acc + tl

    h, c, acc = lax.fori_loop(0, n_dec, dec_step,
                              (h, c, jnp.float32(0.0)))
    tsum_ref[:] = jnp.reshape(acc, (1, 1))


def _run_lstm(enc_g, dec_g, wr_g, br_g, pe, pd, pt, pb,
              wih_i, whh_i, bi, wih_t, whh_t, bt, n_enc, n_dec, batch, hd):
    body = functools.partial(_lstm_body, n_enc=n_enc, n_dec=n_dec,
                             batch=batch, hd=hd)
    return pl.pallas_call(
        body,
        out_shape=[
            jax.ShapeDtypeStruct((n_dec * batch, hd), jnp.bfloat16),
            jax.ShapeDtypeStruct((1, 1), jnp.float32),
        ],
    )(enc_g, dec_g, wr_g, br_g, pe, pd, pt, pb,
      wih_i, whh_i, bi, wih_t, whh_t, bt)


def _lse_body(h_ref, w_ref, b_ref, tsum_ref, out_ref, s_ref, *, num_v, rt,
              inv_batch):
    v = pl.program_id(0)
    r = pl.program_id(1)
    logits = lax.dot_general(
        w_ref[:], h_ref[:], (((1,), (1,)), ((), ())),
        preferred_element_type=jnp.float32)
    part = jnp.sum(jnp.exp2(logits + b_ref[:]), axis=0).reshape(rt // 128, 128)

    @pl.when(jnp.logical_and(v == 0, r == 0))
    def _():
        out_ref[:] = -tsum_ref[:] * inv_batch

    @pl.when(v == 0)
    def _():
        s_ref[r] = part

    @pl.when(v > 0)
    def _():
        s_ref[r] = s_ref[r] + part

    @pl.when(v == num_v - 1)
    def _():
        lse = jnp.log(s_ref[r])
        out_ref[:] = out_ref[:] + jnp.sum(lse, keepdims=True).reshape(1, 1) * inv_batch


def _run_lse(h_bf, w_bf, b_pad, tsum, batch):
    n_rows = h_bf.shape[0]
    vp = w_bf.shape[0]
    vt = _VT
    rt = 1024
    num_v = vp // vt
    num_r = n_rows // rt
    body = functools.partial(_lse_body, num_v=num_v, rt=rt,
                             inv_batch=1.0 / batch)
    return pl.pallas_call(
        body,
        grid=(num_v, num_r),
        in_specs=[
            pl.BlockSpec((rt, h_bf.shape[1]), lambda v, r: (r, 0)),
            pl.BlockSpec((vt, w_bf.shape[1]), lambda v, r: (v, 0)),
            pl.BlockSpec((vt, 1), lambda v, r: (v, 0)),
            pl.BlockSpec((1, 1), lambda v, r: (0, 0)),
        ],
        out_specs=pl.BlockSpec((1, 1), lambda v, r: (0, 0)),
        out_shape=jax.ShapeDtypeStruct((1, 1), jnp.float32),
        scratch_shapes=[pltpu.VMEM((num_r, rt // 128, 128), jnp.float32)],
    )(h_bf, w_bf, b_pad, tsum)


def kernel(input_lines, target_lines, embed_input, embed_target,
           W_ih_in, W_hh_in, b_ih_in, b_hh_in,
           W_ih_tg, W_hh_tg, b_ih_tg, b_hh_tg,
           W_lin, b_lin):
    s_in, batch = input_lines.shape
    s_out = target_lines.shape[0]
    t_dec = s_out - 1
    hd = embed_input.shape[1]
    v = W_lin.shape[0]

    idx_enc = input_lines.reshape(-1)
    idx_dec = target_lines[:t_dec].reshape(-1)
    idx_tgt = target_lines[1:].reshape(-1)

    emb_in2 = embed_input.reshape(embed_input.shape[0] // 2, 2 * hd)
    emb_tg2 = embed_target.reshape(embed_target.shape[0] // 2, 2 * hd)
    w_lin2 = W_lin.reshape(v // 2, 2 * hd)
    b_pad128 = jnp.pad(b_lin, (0, (-v) % 128))
    b_lin2 = b_pad128.reshape(b_pad128.shape[0] // 128, 128)

    pe = jnp.bitwise_and(idx_enc, 1)
    pd = jnp.bitwise_and(idx_dec, 1)
    pt = jnp.bitwise_and(idx_tgt, 1)
    pb = jnp.bitwise_and(idx_tgt, 127)

    enc_g, dec_g, wr_g, br_g = _sc_gather_all(
        emb_in2, emb_tg2, w_lin2, b_lin2,
        lax.shift_right_logical(idx_enc, 1),
        lax.shift_right_logical(idx_dec, 1),
        lax.shift_right_logical(idx_tgt, 1),
        lax.shift_right_logical(idx_tgt, 7))

    h_bf, tsum = _run_lstm(
        enc_g, dec_g, wr_g, br_g, pe, pd, pt, pb,
        W_ih_in.T, W_hh_in.T, (b_ih_in + b_hh_in)[None, :],
        W_ih_tg.T, W_hh_tg.T, (b_ih_tg + b_hh_tg)[None, :],
        s_in, t_dec, batch, hd)

    vt = _VT
    vp = ((v + vt - 1) // vt) * vt
    log2e = 1.4426950408889634
    w_bf = jnp.pad((W_lin * log2e).astype(jnp.bfloat16), ((0, vp - v), (0, 0)))
    b_pad = jnp.pad(b_lin * log2e, (0, vp - v),
                    constant_values=-1e30).reshape(vp, 1)

    loss = _run_lse(h_bf, w_bf, b_pad, tsum, batch)
    return loss.reshape(())
